# baseline (speedup 1.0000x reference)
import functools

import jax
import jax.numpy as jnp
from jax import lax
from jax.experimental import pallas as pl
from jax.experimental.pallas import tpu as pltpu
from jax.experimental.pallas import tpu_sc as plsc

NC = 2
NS = 16
NW = NC * NS
K = 40

_LP = (1.14470976e-05, 0.999166401, -0.48969909, 0.283823183,
       -0.129957198, 0.0298087652)


def _ssp(v):
    z = 0.5 * v
    u = jnp.exp(-jnp.abs(z))
    u2 = u * u
    u4 = u2 * u2
    p = (jnp.float32(_LP[0]) + jnp.float32(_LP[1]) * u
         + (jnp.float32(_LP[2]) + jnp.float32(_LP[3]) * u) * u2
         + (jnp.float32(_LP[4]) + jnp.float32(_LP[5]) * u) * u4)
    return 2.0 * (jnp.maximum(z, 0.0) + p)


def _mm_t(a, b):
    return lax.dot_general(a, b, (((1,), (1,)), ((), ())),
                           preferred_element_type=jnp.float32)


def _nn_body(x_ref, w_ref, o_ref):
    o_ref[...] = _mm_t(x_ref[...], w_ref[...])


def _emlp_body(rbft_ref, wc1_ref, bc1_ref, wc2_ref, bc2_ref, h_ref):
    a = lax.dot_general(rbft_ref[...], wc1_ref[...], (((0,), (1,)), ((), ())),
                        preferred_element_type=jnp.float32)
    a = _ssp(a + bc1_ref[...])
    h_ref[...] = _mm_t(a, wc2_ref[...]) + bc2_ref[...]


def _out_body(p_ref, x_ref, w2_ref, b2_ref, w3_ref, b3_ref, o_ref):
    cf = p_ref[0] + p_ref[1]
    t = _ssp(_mm_t(cf, w2_ref[...]) + b2_ref[...])
    o_ref[...] = x_ref[...] + _mm_t(t, w3_ref[...]) + b3_ref[...]


@functools.cache
def _make_sc(N, D, E):
    EW = E // NW
    CH = EW // K
    RT = (N // NS) // 8 * 8
    TAIL = N - NS * RT
    mesh = plsc.VectorSubcoreMesh(core_axis_name="c", subcore_axis_name="s")

    @functools.partial(
        pl.kernel,
        out_type=jax.ShapeDtypeStruct((NC, N, D), jnp.float32),
        mesh=mesh,
        scratch_types=[
            pltpu.VMEM((EW,), jnp.int32),
            pltpu.VMEM((2, 1, K), jnp.int32),
            pltpu.VMEM((2, K, D), jnp.float32),
            pltpu.VMEM((2, K, D), jnp.float32),
            pltpu.VMEM_SHARED((N, D), jnp.float32),
            pltpu.SemaphoreType.DMA,
            pltpu.SemaphoreType.DMA,
            pltpu.SemaphoreType.DMA,
            pltpu.SemaphoreType.DMA,
            pltpu.SemaphoreType.DMA,
            pltpu.SemaphoreType.DMA,
            pltpu.SemaphoreType.DMA,
            pltpu.SemaphoreType.DMA,
        ],
    )
    def sc_fn(nn_hbm, h_hbm, src_hbm, dst_hbm, zero_hbm, out_hbm,
              src_v, dst_v, gath_v, h_v, acc_sh,
              sem_d0, sem_d1, sem_g0, sem_g1, sem_h0, sem_h1,
              sem_s0, sem_s1):
        cid = lax.axis_index("c")
        sid = lax.axis_index("s")
        wid = cid * NS + sid
        base = wid * EW
        sem_d = (sem_d0, sem_d1)
        sem_g = (sem_g0, sem_g1)
        sem_h = (sem_h0, sem_h1)
        sem_s = (sem_s0, sem_s1)

        pltpu.sync_copy(zero_hbm.at[pl.ds(sid * RT, RT)],
                        acc_sh.at[pl.ds(sid * RT, RT)])
        @pl.when(sid == NS - 1)
        def _():
            pltpu.sync_copy(zero_hbm.at[pl.ds(NS * RT, TAIL)],
                            acc_sh.at[pl.ds(NS * RT, TAIL)])
        pltpu.sync_copy(src_hbm.at[pl.ds(base, EW)], src_v)
        plsc.subcore_barrier()

        def wait_scatter(b):
            pltpu.make_async_copy(gath_v.at[b], acc_sh.at[dst_v.at[b, 0]],
                                  sem_s[b]).wait()

        def issue(c, b):
            @pl.when(c >= 2)
            def _():
                wait_scatter(b)
            pltpu.async_copy(dst_hbm.at[wid * CH + c], dst_v.at[b], sem_d[b])
            pltpu.async_copy(h_hbm.at[pl.ds(base + c * K, K)], h_v.at[b],
                             sem_h[b])
            pltpu.async_copy(nn_hbm.at[src_v.at[pl.ds(c * K, K)]],
                             gath_v.at[b], sem_g[b])

        def process(c, b):
            pltpu.make_async_copy(h_hbm.at[pl.ds(base + c * K, K)],
                                  h_v.at[b], sem_h[b]).wait()
            pltpu.make_async_copy(nn_hbm.at[src_v.at[pl.ds(c * K, K)]],
                                  gath_v.at[b], sem_g[b]).wait()

            def row(r, carry2):
                for j in range(D // 16):
                    s = pl.ds(j * 16, 16)
                    gath_v[b, r, s] = gath_v[b, r, s] * h_v[b, r, s]
                return carry2

            lax.fori_loop(0, K, row, 0, unroll=8)

            pltpu.make_async_copy(dst_hbm.at[wid * CH + c],
                                  dst_v.at[b], sem_d[b]).wait()
            pltpu.async_copy(gath_v.at[b], acc_sh.at[dst_v.at[b, 0]],
                             sem_s[b], add=True)

        issue(0, 0)

        def pair(i, carry):
            c0 = 2 * i
            issue(c0 + 1, 1)
            process(c0, 0)

            @pl.when(i < CH // 2 - 1)
            def _():
                issue(c0 + 2, 0)

            process(c0 + 1, 1)
            return carry

        lax.fori_loop(0, CH // 2, pair, 0)
        wait_scatter(0)
        wait_scatter(1)
        plsc.subcore_barrier()

        pltpu.sync_copy(acc_sh.at[pl.ds(sid * RT, RT)],
                        out_hbm.at[cid, pl.ds(sid * RT, RT)])
        @pl.when(sid == NS - 1)
        def _():
            pltpu.sync_copy(acc_sh.at[pl.ds(NS * RT, TAIL)],
                            out_hbm.at[cid, pl.ds(NS * RT, TAIL)])

    return sc_fn


def kernel(x, edge_index, rbf, W1, Wc1, bc1, Wc2, bc2, W2, b2, W3, b3):
    N, D = x.shape
    E, R = rbf.shape
    EW = E // NW
    CH = EW // K

    new_node = pl.pallas_call(
        _nn_body,
        out_shape=jax.ShapeDtypeStruct((N, D), jnp.float32),
    )(x, W1)

    BE = 4096
    h = pl.pallas_call(
        _emlp_body,
        grid=((E + BE - 1) // BE,),
        in_specs=[
            pl.BlockSpec((R, BE), lambda i: (0, i)),
            pl.BlockSpec((D, R), lambda i: (0, 0)),
            pl.BlockSpec((1, D), lambda i: (0, 0)),
            pl.BlockSpec((D, D), lambda i: (0, 0)),
            pl.BlockSpec((1, D), lambda i: (0, 0)),
        ],
        out_specs=pl.BlockSpec((BE, D), lambda i: (i, 0)),
        out_shape=jax.ShapeDtypeStruct((E, D), jnp.float32),
    )(rbf.T, Wc1, bc1.reshape(1, D), Wc2, bc2.reshape(1, D))

    src = edge_index[0]
    dst3 = edge_index[1].reshape(NW * CH, 1, K)
    zeros = jnp.zeros((N, D), jnp.float32)
    partials = _make_sc(N, D, E)(new_node, h, src, dst3, zeros)

    BN = 2000
    out = pl.pallas_call(
        _out_body,
        grid=(N // BN,),
        in_specs=[
            pl.BlockSpec((NC, BN, D), lambda i: (0, i, 0)),
            pl.BlockSpec((BN, D), lambda i: (i, 0)),
            pl.BlockSpec((D, D), lambda i: (0, 0)),
            pl.BlockSpec((1, D), lambda i: (0, 0)),
            pl.BlockSpec((D, D), lambda i: (0, 0)),
            pl.BlockSpec((1, D), lambda i: (0, 0)),
        ],
        out_specs=pl.BlockSpec((BN, D), lambda i: (i, 0)),
        out_shape=jax.ShapeDtypeStruct((N, D), jnp.float32),
    )(partials, x, W2, b2.reshape(1, D), W3, b3.reshape(1, D))

    return out


# speedup vs baseline: 1.6982x; 1.6982x over previous
import functools

import jax
import jax.numpy as jnp
from jax import lax
from jax.experimental import pallas as pl
from jax.experimental.pallas import tpu as pltpu
from jax.experimental.pallas import tpu_sc as plsc

NC = 2
NS = 16
NW = NC * NS
K = 40

_LP = (1.14470976e-05, 0.999166401, -0.48969909, 0.283823183,
       -0.129957198, 0.0298087652)


def _ssp(v):
    z = 0.5 * v
    u = jnp.exp(-jnp.abs(z))
    u2 = u * u
    u4 = u2 * u2
    p = (jnp.float32(_LP[0]) + jnp.float32(_LP[1]) * u
         + (jnp.float32(_LP[2]) + jnp.float32(_LP[3]) * u) * u2
         + (jnp.float32(_LP[4]) + jnp.float32(_LP[5]) * u) * u4)
    return 2.0 * (jnp.maximum(z, 0.0) + p)


def _mm_t(a, b):
    return lax.dot_general(a, b, (((1,), (1,)), ((), ())),
                           preferred_element_type=jnp.float32)


def _nn_body(x_ref, w_ref, o_ref):
    o_ref[...] = _mm_t(x_ref[...], w_ref[...])


def _emlp_body(rbft_ref, wc1_ref, bc1_ref, wc2_ref, bc2_ref, h_ref):
    a = lax.dot_general(rbft_ref[...], wc1_ref[...], (((0,), (1,)), ((), ())),
                        preferred_element_type=jnp.float32)
    a = _ssp(a + bc1_ref[...])
    h_ref[...] = _mm_t(a, wc2_ref[...]) + bc2_ref[...]


def _out_body(p0_ref, p1_ref, x_ref, w2_ref, b2_ref, w3_ref, b3_ref, o_ref):
    cf = (p0_ref[0] + p0_ref[1]) + (p1_ref[0] + p1_ref[1])
    t = _ssp(_mm_t(cf, w2_ref[...]) + b2_ref[...])
    o_ref[...] = x_ref[...] + _mm_t(t, w3_ref[...]) + b3_ref[...]


@functools.cache
def _make_sc(N, D, E):
    EW = E // NW
    CH = EW // K
    RT = (N // NS) // 8 * 8
    TAIL = N - NS * RT
    mesh = plsc.VectorSubcoreMesh(core_axis_name="c", subcore_axis_name="s")

    @functools.partial(
        pl.kernel,
        out_type=jax.ShapeDtypeStruct((NC, N, D), jnp.float32),
        mesh=mesh,
        scratch_types=[
            pltpu.VMEM((EW,), jnp.int32),
            pltpu.VMEM((2, 1, K), jnp.int32),
            pltpu.VMEM((2, K, D), jnp.float32),
            pltpu.VMEM((2, K, D), jnp.float32),
            pltpu.VMEM_SHARED((N, D), jnp.float32),
            pltpu.SemaphoreType.DMA,
            pltpu.SemaphoreType.DMA,
            pltpu.SemaphoreType.DMA,
            pltpu.SemaphoreType.DMA,
            pltpu.SemaphoreType.DMA,
            pltpu.SemaphoreType.DMA,
        ],
    )
    def sc_fn(nn_hbm, h_hbm, src_hbm, dst_hbm, zero_hbm, out_hbm,
              src_v, dst_v, gath_v, h_v, acc_sh,
              sem_d0, sem_d1, sem_g0, sem_g1, sem_h0, sem_h1):
        cid = lax.axis_index("c")
        sid = lax.axis_index("s")
        wid = cid * NS + sid
        base = wid * EW
        sem_d = (sem_d0, sem_d1)
        sem_g = (sem_g0, sem_g1)
        sem_h = (sem_h0, sem_h1)

        pltpu.sync_copy(zero_hbm.at[pl.ds(sid * RT, RT)],
                        acc_sh.at[pl.ds(sid * RT, RT)])
        @pl.when(sid == NS - 1)
        def _():
            pltpu.sync_copy(zero_hbm.at[pl.ds(NS * RT, TAIL)],
                            acc_sh.at[pl.ds(NS * RT, TAIL)])
        pltpu.sync_copy(src_hbm.at[pl.ds(base, EW)], src_v)
        plsc.subcore_barrier()

        def issue(c, b):
            pltpu.async_copy(dst_hbm.at[wid * CH + c], dst_v.at[b], sem_d[b])
            pltpu.async_copy(h_hbm.at[pl.ds(base + c * K, K)], h_v.at[b],
                             sem_h[b])
            pltpu.async_copy(nn_hbm.at[src_v.at[pl.ds(c * K, K)]],
                             gath_v.at[b], sem_g[b])

        def process(c, b):
            pltpu.make_async_copy(h_hbm.at[pl.ds(base + c * K, K)],
                                  h_v.at[b], sem_h[b]).wait()
            pltpu.make_async_copy(nn_hbm.at[src_v.at[pl.ds(c * K, K)]],
                                  gath_v.at[b], sem_g[b]).wait()

            def row(r, carry2):
                for j in range(D // 16):
                    s = pl.ds(j * 16, 16)
                    gath_v[b, r, s] = gath_v[b, r, s] * h_v[b, r, s]
                return carry2

            lax.fori_loop(0, K, row, 0)

            pltpu.make_async_copy(dst_hbm.at[wid * CH + c],
                                  dst_v.at[b], sem_d[b]).wait()
            pltpu.sync_copy(gath_v.at[b], acc_sh.at[dst_v.at[b, 0]], add=True)

        issue(0, 0)

        def pair(i, carry):
            c0 = 2 * i
            issue(c0 + 1, 1)
            process(c0, 0)

            @pl.when(c0 + 2 < CH)
            def _():
                issue(c0 + 2, 0)

            process(c0 + 1, 1)
            return carry

        lax.fori_loop(0, CH // 2, pair, 0)
        if CH % 2:
            process(CH - 1, 0)
        plsc.subcore_barrier()

        pltpu.sync_copy(acc_sh.at[pl.ds(sid * RT, RT)],
                        out_hbm.at[cid, pl.ds(sid * RT, RT)])
        @pl.when(sid == NS - 1)
        def _():
            pltpu.sync_copy(acc_sh.at[pl.ds(NS * RT, TAIL)],
                            out_hbm.at[cid, pl.ds(NS * RT, TAIL)])

    return sc_fn


def kernel(x, edge_index, rbf, W1, Wc1, bc1, Wc2, bc2, W2, b2, W3, b3):
    N, D = x.shape
    E, R = rbf.shape
    EW = E // NW
    CH = EW // K

    new_node = pl.pallas_call(
        _nn_body,
        out_shape=jax.ShapeDtypeStruct((N, D), jnp.float32),
    )(x, W1)

    S = 2
    ES = E // S
    BE = 6400
    rbf_t = rbf.T
    src = edge_index[0]
    dst = edge_index[1]
    zeros = jnp.zeros((N, D), jnp.float32)
    sc_call = _make_sc(N, D, ES)

    partials = []
    for s in range(S):
        h_s = pl.pallas_call(
            _emlp_body,
            grid=(ES // BE,),
            in_specs=[
                pl.BlockSpec((R, BE), lambda i, s=s: (0, s * (ES // BE) + i)),
                pl.BlockSpec((D, R), lambda i: (0, 0)),
                pl.BlockSpec((1, D), lambda i: (0, 0)),
                pl.BlockSpec((D, D), lambda i: (0, 0)),
                pl.BlockSpec((1, D), lambda i: (0, 0)),
            ],
            out_specs=pl.BlockSpec((BE, D), lambda i: (i, 0)),
            out_shape=jax.ShapeDtypeStruct((ES, D), jnp.float32),
        )(rbf_t, Wc1, bc1.reshape(1, D), Wc2, bc2.reshape(1, D))
        src_s = lax.slice(src, (s * ES,), ((s + 1) * ES,))
        dst_s = lax.slice(dst, (s * ES,), ((s + 1) * ES,)).reshape(-1, 1, K)
        partials.append(sc_call(new_node, h_s, src_s, dst_s, zeros))

    BN = 2000
    out = pl.pallas_call(
        _out_body,
        grid=(N // BN,),
        in_specs=[
            pl.BlockSpec((NC, BN, D), lambda i: (0, i, 0)),
            pl.BlockSpec((NC, BN, D), lambda i: (0, i, 0)),
            pl.BlockSpec((BN, D), lambda i: (i, 0)),
            pl.BlockSpec((D, D), lambda i: (0, 0)),
            pl.BlockSpec((1, D), lambda i: (0, 0)),
            pl.BlockSpec((D, D), lambda i: (0, 0)),
            pl.BlockSpec((1, D), lambda i: (0, 0)),
        ],
        out_specs=pl.BlockSpec((BN, D), lambda i: (i, 0)),
        out_shape=jax.ShapeDtypeStruct((N, D), jnp.float32),
    )(partials[0], partials[1], x, W2, b2.reshape(1, D),
      W3, b3.reshape(1, D))

    return out
